# baseline (speedup 1.0000x reference)
import functools

import jax
import jax.numpy as jnp
from jax import lax
from jax.experimental import pallas as pl
from jax.experimental.pallas import tpu as pltpu
from jax.experimental.pallas import tpu_sc as plsc

B, T, D = 16, 2048, 256
VOCAB = 100000

_NC = 2
_NS = 16
_LANES = 16
_NW = _NC * _NS
_N_IDX = B * T
_CHUNK = _N_IDX // _NW


def _sc_gather(weights, idx_flat):
    mesh = plsc.VectorSubcoreMesh(core_axis_name="c", subcore_axis_name="s")

    @functools.partial(
        pl.kernel,
        mesh=mesh,
        out_type=jax.ShapeDtypeStruct((_N_IDX,), jnp.float32),
        scratch_types=[
            pltpu.VMEM((VOCAB,), jnp.float32),
            pltpu.VMEM((_CHUNK,), jnp.int32),
            pltpu.VMEM((_CHUNK,), jnp.float32),
        ],
        compiler_params=pltpu.CompilerParams(needs_layout_passes=False),
    )
    def gather_kernel(w_hbm, idx_hbm, out_hbm, wtab_v, idx_v, out_v):
        wid = lax.axis_index("s") * _NC + lax.axis_index("c")
        base = wid * _CHUNK
        pltpu.sync_copy(w_hbm, wtab_v)
        pltpu.sync_copy(idx_hbm.at[pl.ds(base, _CHUNK)], idx_v)

        def body(i, carry):
            off = i * _LANES
            idx16 = idx_v[pl.ds(off, _LANES)]
            out_v[pl.ds(off, _LANES)] = plsc.load_gather(wtab_v, [idx16])
            return carry

        lax.fori_loop(0, _CHUNK // _LANES, body, 0, unroll=4)
        pltpu.sync_copy(out_v, out_hbm.at[pl.ds(base, _CHUNK)])

    return gather_kernel(weights, idx_flat)


_TC = 512
_NT = T // _TC


def _tc_body(len_ref, vs_ref, w_ref, y_ref, yh_ref, acc_ref):
    b = pl.program_id(0)
    t = pl.program_id(1)
    length = len_ref[b]
    pos = lax.broadcasted_iota(jnp.int32, (1, _TC), 1) + t * _TC
    maskf = (pos < length).astype(jnp.float32)
    w_row = w_ref[0, 0, :, :] * maskf
    a = jnp.concatenate([maskf, w_row], axis=0)
    vs = vs_ref[0, :, :]
    part = jnp.dot(a, vs, preferred_element_type=jnp.float32)

    @pl.when(t == 0)
    def _():
        acc_ref[...] = jnp.zeros_like(acc_ref)

    acc_ref[...] += part

    @pl.when(t == _NT - 1)
    def _():
        s = acc_ref[0:1, :]
        denom = jnp.sqrt(jnp.sum(jnp.abs(s)))
        y_ref[0, :, :] = s / denom
        yh_ref[0, :, :] = acc_ref[1:2, :]


def kernel(vector_sequence, sentence_length, word_sequence, weights):
    idx_flat = word_sequence.reshape(-1).astype(jnp.int32)
    w_tok = _sc_gather(weights, idx_flat)
    w4 = w_tok.reshape(B, _NT, 1, _TC)
    lens = sentence_length.astype(jnp.int32)

    y3, yh3 = pl.pallas_call(
        _tc_body,
        grid=(B, _NT),
        in_specs=[
            pl.BlockSpec(memory_space=pltpu.SMEM),
            pl.BlockSpec((1, _TC, D), lambda b, t: (b, t, 0)),
            pl.BlockSpec((1, 1, 1, _TC), lambda b, t: (b, t, 0, 0)),
        ],
        out_specs=[
            pl.BlockSpec((1, 1, D), lambda b, t: (b, 0, 0)),
            pl.BlockSpec((1, 1, D), lambda b, t: (b, 0, 0)),
        ],
        out_shape=[
            jax.ShapeDtypeStruct((B, 1, D), jnp.float32),
            jax.ShapeDtypeStruct((B, 1, D), jnp.float32),
        ],
        scratch_shapes=[pltpu.VMEM((2, D), jnp.float32)],
    )(lens, vector_sequence, w4)
    return y3.reshape(B, D), yh3.reshape(B, D)


# speedup vs baseline: 1.4934x; 1.4934x over previous
import functools

import jax
import jax.numpy as jnp
from jax import lax
from jax.experimental import pallas as pl
from jax.experimental.pallas import tpu as pltpu
from jax.experimental.pallas import tpu_sc as plsc

B, T, D = 16, 2048, 256
VOCAB = 100000

_NC = 2
_NS = 16
_LANES = 16
_NW = _NC * _NS
_N_IDX = B * T
_CHUNK = _N_IDX // _NW


def _sc_gather(weights, idx_flat):
    mesh = plsc.VectorSubcoreMesh(core_axis_name="c", subcore_axis_name="s")

    @functools.partial(
        pl.kernel,
        mesh=mesh,
        out_type=jax.ShapeDtypeStruct((_N_IDX,), jnp.float32),
        scratch_types=[
            pltpu.VMEM((VOCAB,), jnp.float32),
            pltpu.VMEM((_CHUNK,), jnp.int32),
            pltpu.VMEM((_CHUNK,), jnp.float32),
        ],
        compiler_params=pltpu.CompilerParams(needs_layout_passes=False),
    )
    def gather_kernel(w_hbm, idx_hbm, out_hbm, wtab_v, idx_v, out_v):
        wid = lax.axis_index("s") * _NC + lax.axis_index("c")
        base = wid * _CHUNK
        pltpu.sync_copy(w_hbm, wtab_v)
        pltpu.sync_copy(idx_hbm.at[pl.ds(base, _CHUNK)], idx_v)

        def body(i, carry):
            off = i * _LANES
            idx16 = idx_v[pl.ds(off, _LANES)]
            out_v[pl.ds(off, _LANES)] = plsc.load_gather(wtab_v, [idx16])
            return carry

        lax.fori_loop(0, _CHUNK // _LANES, body, 0, unroll=4)
        pltpu.sync_copy(out_v, out_hbm.at[pl.ds(base, _CHUNK)])

    return gather_kernel(weights, idx_flat)


_NSPLIT = 4
_TC = T // _NSPLIT


def _tc_body(len_ref, *refs):
    vs_refs = refs[:_NSPLIT]
    w_ref = refs[_NSPLIT]
    y_ref, yh_ref = refs[_NSPLIT + 1], refs[_NSPLIT + 2]
    b = pl.program_id(0)
    length = len_ref[b]
    pos = lax.broadcasted_iota(jnp.int32, (1, T), 1)
    maskf = (pos < length).astype(jnp.float32)
    w_row = w_ref[0] * maskf
    a = jnp.concatenate([maskf, w_row], axis=0)
    acc = jnp.zeros((2, D), jnp.float32)
    for q in range(_NSPLIT):
        a_q = a[:, q * _TC:(q + 1) * _TC]
        acc = acc + jnp.dot(a_q, vs_refs[q][0], preferred_element_type=jnp.float32)
    s = acc[0:1, :]
    denom = jnp.sqrt(jnp.sum(jnp.abs(s)))
    y_ref[0, :, :] = s / denom
    yh_ref[0, :, :] = acc[1:2, :]


def kernel(vector_sequence, sentence_length, word_sequence, weights):
    idx_flat = word_sequence.reshape(-1).astype(jnp.int32)
    w_tok = _sc_gather(weights, idx_flat)
    w3 = w_tok.reshape(B, 1, T)
    lens = sentence_length.astype(jnp.int32)
    vs4 = vector_sequence.reshape(B, _NSPLIT, _TC, D)

    vs_specs = [
        pl.BlockSpec((1, 1, _TC, D), functools.partial(lambda q, b: (b, q, 0, 0), q))
        for q in range(_NSPLIT)
    ]

    def _squeeze_body(len_ref, *refs):
        vs_refs = [r.at[0] for r in refs[:_NSPLIT]]
        _tc_body(len_ref, *vs_refs, *refs[_NSPLIT:])

    y3, yh3 = pl.pallas_call(
        _squeeze_body,
        grid=(B,),
        in_specs=[
            pl.BlockSpec(memory_space=pltpu.SMEM),
            *vs_specs,
            pl.BlockSpec((1, 1, T), lambda b: (b, 0, 0)),
        ],
        out_specs=[
            pl.BlockSpec((1, 1, D), lambda b: (b, 0, 0)),
            pl.BlockSpec((1, 1, D), lambda b: (b, 0, 0)),
        ],
        out_shape=[
            jax.ShapeDtypeStruct((B, 1, D), jnp.float32),
            jax.ShapeDtypeStruct((B, 1, D), jnp.float32),
        ],
    )(lens, *([vs4] * _NSPLIT), w3)
    return y3.reshape(B, D), yh3.reshape(B, D)


# speedup vs baseline: 1.6225x; 1.0865x over previous
import functools

import jax
import jax.numpy as jnp
from jax import lax
from jax.experimental import pallas as pl
from jax.experimental.pallas import tpu as pltpu
from jax.experimental.pallas import tpu_sc as plsc

B, T, D = 16, 2048, 256
VOCAB = 100000

_NC = 2
_NS = 16
_LANES = 16
_NW = _NC * _NS
_N_IDX = B * T
_CHUNK = _N_IDX // _NW


def _sc_gather(weights, idx_flat):
    mesh = plsc.VectorSubcoreMesh(core_axis_name="c", subcore_axis_name="s")

    @functools.partial(
        pl.kernel,
        mesh=mesh,
        out_type=jax.ShapeDtypeStruct((_N_IDX,), jnp.float32),
        scratch_types=[
            pltpu.VMEM((VOCAB,), jnp.float32),
            pltpu.VMEM((_CHUNK,), jnp.int32),
            pltpu.VMEM((_CHUNK,), jnp.float32),
        ],
        compiler_params=pltpu.CompilerParams(needs_layout_passes=False),
    )
    def gather_kernel(w_hbm, idx_hbm, out_hbm, wtab_v, idx_v, out_v):
        wid = lax.axis_index("s") * _NC + lax.axis_index("c")
        base = wid * _CHUNK
        pltpu.sync_copy(w_hbm, wtab_v)
        pltpu.sync_copy(idx_hbm.at[pl.ds(base, _CHUNK)], idx_v)

        def body(i, carry):
            off = i * _LANES
            idx16 = idx_v[pl.ds(off, _LANES)]
            out_v[pl.ds(off, _LANES)] = plsc.load_gather(wtab_v, [idx16])
            return carry

        lax.fori_loop(0, _CHUNK // _LANES, body, 0, unroll=4)
        pltpu.sync_copy(out_v, out_hbm.at[pl.ds(base, _CHUNK)])

    return gather_kernel(weights, idx_flat)


_HB = B // 2


def _one_row(length, vs, w_row_raw, y_ref, yh_ref):
    pos = lax.broadcasted_iota(jnp.int32, (1, T), 1)
    maskf = (pos < length).astype(jnp.float32)
    w_row = w_row_raw * maskf
    a = jnp.concatenate([maskf, w_row], axis=0)
    acc = jnp.dot(a, vs, preferred_element_type=jnp.float32)
    s = acc[0:1, :]
    denom = jnp.sqrt(jnp.sum(jnp.abs(s)))
    y_ref[0, :, :] = s / denom
    yh_ref[0, :, :] = acc[1:2, :]


def _tc_body(len_ref, vs_lo, vs_hi, w_lo, w_hi, y_lo, y_hi, yh_lo, yh_hi):
    b = pl.program_id(0)
    _one_row(len_ref[b], vs_lo[0], w_lo[0], y_lo, yh_lo)
    _one_row(len_ref[b + _HB], vs_hi[0], w_hi[0], y_hi, yh_hi)


def kernel(vector_sequence, sentence_length, word_sequence, weights):
    idx_flat = word_sequence.reshape(-1).astype(jnp.int32)
    w_tok = _sc_gather(weights, idx_flat)
    w3 = w_tok.reshape(B, 1, T)
    lens = sentence_length.astype(jnp.int32)

    out_spec = pl.BlockSpec((1, 1, D), lambda b: (b, 0, 0))
    out_ty = jax.ShapeDtypeStruct((_HB, 1, D), jnp.float32)
    y_lo, y_hi, yh_lo, yh_hi = pl.pallas_call(
        _tc_body,
        grid=(_HB,),
        in_specs=[
            pl.BlockSpec(memory_space=pltpu.SMEM),
            pl.BlockSpec((1, T, D), lambda b: (b, 0, 0)),
            pl.BlockSpec((1, T, D), lambda b: (b + _HB, 0, 0)),
            pl.BlockSpec((1, 1, T), lambda b: (b, 0, 0)),
            pl.BlockSpec((1, 1, T), lambda b: (b + _HB, 0, 0)),
        ],
        out_specs=[out_spec] * 4,
        out_shape=[out_ty] * 4,
    )(lens, vector_sequence, vector_sequence, w3, w3)
    y = jnp.concatenate([y_lo, y_hi], axis=0).reshape(B, D)
    y_hat = jnp.concatenate([yh_lo, yh_hi], axis=0).reshape(B, D)
    return y, y_hat


# speedup vs baseline: 2.0453x; 1.2606x over previous
import functools

import jax
import jax.numpy as jnp
from jax import lax
from jax.experimental import pallas as pl
from jax.experimental.pallas import tpu as pltpu
from jax.experimental.pallas import tpu_sc as plsc

B, T, D = 16, 2048, 256
VOCAB = 100000

_NC = 2
_NS = 16
_LANES = 16
_NW = _NC * _NS
_N_IDX = B * T
_CHUNK = _N_IDX // _NW


_SUB = 8
_SUBW = _CHUNK // _SUB


def _sc_gather(weights, idx3):
    mesh = plsc.VectorSubcoreMesh(core_axis_name="c", subcore_axis_name="s")

    @functools.partial(
        pl.kernel,
        mesh=mesh,
        out_type=jax.ShapeDtypeStruct((_NW, _SUB, _SUBW), jnp.float32),
        scratch_types=[
            pltpu.VMEM((_SUB, _SUBW), jnp.int32),
            pltpu.VMEM((_SUB, _SUBW), jnp.float32),
            pltpu.SemaphoreType.DMA,
        ],
        compiler_params=pltpu.CompilerParams(needs_layout_passes=False),
    )
    def gather_kernel(w_hbm, idx_hbm, out_hbm, idx_v, rows_v, sem):
        wid = lax.axis_index("s") * _NC + lax.axis_index("c")
        pltpu.sync_copy(idx_hbm.at[wid], idx_v)
        copies = [
            pltpu.async_copy(w_hbm.at[idx_v.at[j]], rows_v.at[j], sem)
            for j in range(_SUB)
        ]
        for c in copies:
            c.wait()
        pltpu.sync_copy(rows_v, out_hbm.at[wid])

    return gather_kernel(weights, idx3)


_HB = B // 2


def _one_row(length, vs, w_row_raw, y_ref, yh_ref):
    pos = lax.broadcasted_iota(jnp.int32, (1, T), 1)
    maskf = (pos < length).astype(jnp.float32)
    w_row = w_row_raw * maskf
    a = jnp.concatenate([maskf, w_row], axis=0)
    acc = jnp.dot(a, vs, preferred_element_type=jnp.float32)
    s = acc[0:1, :]
    denom = jnp.sqrt(jnp.sum(jnp.abs(s)))
    y_ref[0, :, :] = s / denom
    yh_ref[0, :, :] = acc[1:2, :]


def _tc_body(len_ref, vs_lo, vs_hi, w_lo, w_hi, y_lo, y_hi, yh_lo, yh_hi):
    b = pl.program_id(0)
    _one_row(len_ref[b], vs_lo[0], w_lo[0], y_lo, yh_lo)
    _one_row(len_ref[b + _HB], vs_hi[0], w_hi[0], y_hi, yh_hi)


def kernel(vector_sequence, sentence_length, word_sequence, weights):
    idx3 = word_sequence.astype(jnp.int32).reshape(_NW, _SUB, _SUBW)
    w_tok = _sc_gather(weights, idx3)
    w3 = w_tok.reshape(B, 1, T)
    lens = sentence_length.astype(jnp.int32)

    out_spec = pl.BlockSpec((1, 1, D), lambda b: (b, 0, 0))
    out_ty = jax.ShapeDtypeStruct((_HB, 1, D), jnp.float32)
    y_lo, y_hi, yh_lo, yh_hi = pl.pallas_call(
        _tc_body,
        grid=(_HB,),
        in_specs=[
            pl.BlockSpec(memory_space=pltpu.SMEM),
            pl.BlockSpec((1, T, D), lambda b: (b, 0, 0)),
            pl.BlockSpec((1, T, D), lambda b: (b + _HB, 0, 0)),
            pl.BlockSpec((1, 1, T), lambda b: (b, 0, 0)),
            pl.BlockSpec((1, 1, T), lambda b: (b + _HB, 0, 0)),
        ],
        out_specs=[out_spec] * 4,
        out_shape=[out_ty] * 4,
    )(lens, vector_sequence, vector_sequence, w3, w3)
    y = jnp.concatenate([y_lo, y_hi], axis=0).reshape(B, D)
    y_hat = jnp.concatenate([yh_lo, yh_hi], axis=0).reshape(B, D)
    return y, y_hat


# speedup vs baseline: 2.1247x; 1.0388x over previous
import functools

import jax
import jax.numpy as jnp
from jax import lax
from jax.experimental import pallas as pl
from jax.experimental.pallas import tpu as pltpu
from jax.experimental.pallas import tpu_sc as plsc

B, T, D = 16, 2048, 256
VOCAB = 100000

_NC = 2
_NS = 16
_LANES = 16
_NW = _NC * _NS
_N_IDX = B * T
_CHUNK = _N_IDX // _NW


_SUB = 8
_SUBW = _CHUNK // _SUB


def _sc_gather(weights, idx3):
    mesh = plsc.VectorSubcoreMesh(core_axis_name="c", subcore_axis_name="s")

    @functools.partial(
        pl.kernel,
        mesh=mesh,
        out_type=jax.ShapeDtypeStruct((_NW, _SUB, _SUBW), jnp.float32),
        scratch_types=[
            pltpu.VMEM((_SUB, _SUBW), jnp.int32),
            pltpu.VMEM((_SUB, _SUBW), jnp.float32),
            pltpu.SemaphoreType.DMA,
        ],
        compiler_params=pltpu.CompilerParams(needs_layout_passes=False),
    )
    def gather_kernel(w_hbm, idx_hbm, out_hbm, idx_v, rows_v, sem):
        wid = lax.axis_index("s") * _NC + lax.axis_index("c")
        pltpu.sync_copy(idx_hbm.at[wid], idx_v)
        copies = [
            pltpu.async_copy(w_hbm.at[idx_v.at[j]], rows_v.at[j], sem)
            for j in range(_SUB)
        ]
        for c in copies:
            c.wait()
        pltpu.sync_copy(rows_v, out_hbm.at[wid])

    return gather_kernel(weights, idx3)


_NROW = 4
_GB = B // _NROW


def _one_row(length, vs, w_row_raw, y_ref, yh_ref):
    pos = lax.broadcasted_iota(jnp.int32, (1, T), 1)
    maskf = (pos < length).astype(jnp.float32)
    w_row = w_row_raw * maskf
    a = jnp.concatenate([maskf, w_row], axis=0)
    acc = jnp.dot(a, vs, preferred_element_type=jnp.float32)
    s = acc[0:1, :]
    denom = jnp.sqrt(jnp.sum(jnp.abs(s)))
    y_ref[0, :, :] = s / denom
    yh_ref[0, :, :] = acc[1:2, :]


def _tc_body(len_ref, *refs):
    vs_refs = refs[:_NROW]
    w_refs = refs[_NROW:2 * _NROW]
    y_refs = refs[2 * _NROW:3 * _NROW]
    yh_refs = refs[3 * _NROW:]
    b = pl.program_id(0)
    for k in range(_NROW):
        _one_row(len_ref[b + k * _GB], vs_refs[k][0], w_refs[k][0],
                 y_refs[k], yh_refs[k])


def kernel(vector_sequence, sentence_length, word_sequence, weights):
    idx3 = word_sequence.astype(jnp.int32).reshape(_NW, _SUB, _SUBW)
    w_tok = _sc_gather(weights, idx3)
    w3 = w_tok.reshape(B, 1, T)
    lens = sentence_length.astype(jnp.int32)

    def _off(k):
        return lambda b: (b + k * _GB, 0, 0)

    vs_specs = [pl.BlockSpec((1, T, D), _off(k)) for k in range(_NROW)]
    w_specs = [pl.BlockSpec((1, 1, T), _off(k)) for k in range(_NROW)]
    out_spec = pl.BlockSpec((1, 1, D), lambda b: (b, 0, 0))
    out_ty = jax.ShapeDtypeStruct((_GB, 1, D), jnp.float32)
    outs = pl.pallas_call(
        _tc_body,
        grid=(_GB,),
        in_specs=[
            pl.BlockSpec(memory_space=pltpu.SMEM),
            *vs_specs,
            *w_specs,
        ],
        out_specs=[out_spec] * (2 * _NROW),
        out_shape=[out_ty] * (2 * _NROW),
    )(lens, *([vector_sequence] * _NROW), *([w3] * _NROW))
    y = jnp.concatenate(outs[:_NROW], axis=0).reshape(B, D)
    y_hat = jnp.concatenate(outs[_NROW:], axis=0).reshape(B, D)
    return y, y_hat
